# baseline (speedup 1.0000x reference)
import functools

import jax
import jax.numpy as jnp
from jax import lax
from jax.experimental import pallas as pl
from jax.experimental.pallas import tpu as pltpu
from jax.experimental.pallas import tpu_sc as plsc

KNN = 20
NPTS = 4096
TN = 256
TNB = 128
TNF = 512
CH = 32
CHC = 128
YW = 128
NWORK = 32
NEG = -3.0e38
BF = jnp.bfloat16


NCL = 128
NSG = NPTS // NCL
RND = 4


def _topk_body(xT_ref, xr_ref, idx_ref, D_ref, cand_ref, cpos_ref):
    b = pl.program_id(0)
    xT = xT_ref[0]
    xr = xr_ref[0]
    G = lax.dot_general(xr.astype(BF), xT.astype(BF), (((1,), (0,)), ((), ())),
                        preferred_element_type=jnp.float32)
    nb = jnp.sum(xT * xT, axis=0, keepdims=True)
    nr = jnp.sum(xr * xr, axis=1, keepdims=True)
    D_ref[...] = 2.0 * G - nr - nb

    gcol = (lax.broadcasted_iota(jnp.int32, (TN, NSG, NCL), 1) * NCL
            + lax.broadcasted_iota(jnp.int32, (TN, NSG, NCL), 2))
    rowsK = lax.broadcasted_iota(jnp.int32, (KNN, TN), 0)
    lanes = lax.broadcasted_iota(jnp.int32, (TN, NPTS), 1)

    def rbody(r, mth):
        D3 = D_ref[...].reshape(TN, NSG, NCL)
        masked = jnp.where(D3 < mth[:, None, :], D3, NEG)
        cm = jnp.max(masked, axis=1)
        pos = jnp.min(jnp.where(masked == cm[:, None, :], gcol, NPTS),
                      axis=1)
        cand_ref[pl.ds(r, 1)] = cm[None]
        cpos_ref[pl.ds(r, 1)] = pos[None]
        return cm

    mth0 = jnp.full((TN, NCL), 3.0e38, jnp.float32)
    mth = lax.fori_loop(0, RND, rbody, mth0)
    g = jnp.max(mth, axis=1, keepdims=True)

    idx_ref[0] = jnp.zeros((KNN, TN), jnp.int32)
    cnd0 = cand_ref[...]
    m0 = jnp.max(jnp.max(cnd0, axis=0), axis=1, keepdims=True)

    def fstep(t, carry):
        m, _ = carry
        cnd = cand_ref[...]
        cp = cpos_ref[...]
        col = jnp.min(jnp.min(jnp.where(cnd == m[None], cp, NPTS), axis=0),
                      axis=1)
        idx_ref[0] = jnp.where(rowsK == t, (col + b * NPTS)[None, :],
                               idx_ref[0])
        nxt = jnp.max(jnp.max(jnp.where(cnd < m[None], cnd, NEG), axis=0),
                      axis=1, keepdims=True)
        return nxt, m

    _, v20 = lax.fori_loop(0, KNN, fstep, (m0, m0))

    @pl.when(jnp.any(v20 < g))
    def _fallback():
        m00 = jnp.max(D_ref[...], axis=1, keepdims=True)

        def step(t, m):
            D = D_ref[...]
            col = jnp.min(jnp.where(D == m, lanes, NPTS), axis=1)
            idx_ref[0] = jnp.where(rowsK == t, (col + b * NPTS)[None, :],
                                   idx_ref[0])
            return jnp.max(jnp.where(D < m, D, NEG), axis=1, keepdims=True)

        lax.fori_loop(0, KNN, step, m00)


def _topk(xr, xT):
    Bn, Nn, C = xr.shape
    return pl.pallas_call(
        _topk_body,
        grid=(Bn, Nn // TN),
        in_specs=[
            pl.BlockSpec((1, C, Nn), lambda b, t: (b, 0, 0)),
            pl.BlockSpec((1, TN, C), lambda b, t: (b, t, 0)),
        ],
        out_specs=pl.BlockSpec((1, KNN, TN), lambda b, t: (b, 0, t)),
        out_shape=jax.ShapeDtypeStruct((Bn, KNN, Nn), jnp.int32),
        scratch_shapes=[pltpu.VMEM((TN, Nn), jnp.float32),
                        pltpu.VMEM((RND, TN, NCL), jnp.float32),
                        pltpu.VMEM((RND, TN, NCL), jnp.int32)],
    )(xT, xr)


def _sc_gather(table, idxg, Bn, Nn):
    P = Bn * Nn
    ppw = P // NWORK
    nslab = ppw // CHC

    mesh = plsc.VectorSubcoreMesh(core_axis_name="c", subcore_axis_name="s")

    @functools.partial(
        pl.kernel,
        mesh=mesh,
        out_type=jax.ShapeDtypeStruct((Bn, KNN, Nn, YW), jnp.float32),
        scratch_types=[
            pltpu.VMEM((KNN, CHC), jnp.int32),
            pltpu.VMEM((KNN, CH, YW), jnp.float32),
            pltpu.SemaphoreType.DMA,
        ],
    )
    def sc_body(tab_hbm, idx_hbm, out_hbm, idxv, rows, sem):
        wid = lax.axis_index("s") * 2 + lax.axis_index("c")
        p0w = wid * ppw
        b = p0w // Nn

        def slab(ci, _):
            jb = p0w + ci * CHC - b * Nn
            pltpu.sync_copy(idx_hbm.at[b, :, pl.ds(jb, CHC)], idxv)

            def sub(h, _):
                jc = jb + h * CH
                cps = [pltpu.async_copy(
                           tab_hbm.at[idxv.at[t, pl.ds(h * CH, CH)]],
                           rows.at[t], sem)
                       for t in range(KNN)]
                for cp in cps:
                    cp.wait()
                pltpu.sync_copy(rows, out_hbm.at[b, :, pl.ds(jc, CH)])
                return 0

            lax.fori_loop(0, CHC // CH, sub, 0)
            return 0

        lax.fori_loop(0, nslab, slab, 0)

    return sc_body(table, idxg)


def _edge_body2(E_ref, xr_ref, W1_ref, b1_ref, W2_ref, b2_ref, o_ref):
    C = xr_ref.shape[2]
    xj = E_ref[0].reshape(KNN * TNB, YW)[:, :C]
    xi = jnp.broadcast_to(xr_ref[0][None], (KNN, TNB, C))
    xi = xi.reshape(KNN * TNB, C)
    feat = jnp.concatenate([xj - xi, xi], axis=1).astype(BF)
    h = lax.dot_general(feat, W1_ref[...], (((1,), (0,)), ((), ())),
                        preferred_element_type=jnp.float32) + b1_ref[...]
    h = lax.dot_general(h.astype(BF), W2_ref[...], (((1,), (0,)), ((), ())),
                        preferred_element_type=jnp.float32) + b2_ref[...]
    o_ref[0] = jnp.max(h.reshape(KNN, TNB, 64), axis=0)


def _edge_body1(E_ref, xr_ref, W1_ref, b1_ref, o_ref):
    C = xr_ref.shape[2]
    xj = E_ref[0].reshape(KNN * TNB, YW)[:, :C]
    xi = jnp.broadcast_to(xr_ref[0][None], (KNN, TNB, C))
    xi = xi.reshape(KNN * TNB, C)
    feat = jnp.concatenate([xj - xi, xi], axis=1).astype(BF)
    h = lax.dot_general(feat, W1_ref[...], (((1,), (0,)), ((), ())),
                        preferred_element_type=jnp.float32) + b1_ref[...]
    o_ref[0] = jnp.max(h.reshape(KNN, TNB, 64), axis=0)


def _edgeconv(E, xr, Ws):
    Bn, Nn, C = xr.shape
    C2 = 2 * C
    wspecs = []
    wargs = []
    for (W, b) in Ws:
        cdim = W.shape[1]
        wspecs += [pl.BlockSpec((cdim, 64), lambda b_, t: (0, 0)),
                   pl.BlockSpec((1, 64), lambda b_, t: (0, 0))]
        wargs += [W.T.astype(BF), b.reshape(1, 64)]
    body = _edge_body2 if len(Ws) == 2 else _edge_body1
    return pl.pallas_call(
        body,
        grid=(Bn, Nn // TNB),
        in_specs=[
            pl.BlockSpec((1, KNN, TNB, YW), lambda b, t: (b, 0, t, 0)),
            pl.BlockSpec((1, TNB, C), lambda b, t: (b, t, 0)),
        ] + wspecs,
        out_specs=pl.BlockSpec((1, TNB, 64), lambda b, t: (b, t, 0)),
        out_shape=jax.ShapeDtypeStruct((Bn, Nn, 64), jnp.float32),
    )(E, xr, *wargs)


def _stage(xr, xT, table, Ws):
    Bn, Nn, _ = xr.shape
    idx = _topk(xr, xT)
    E = _sc_gather(table, idx, Bn, Nn)
    return _edgeconv(E, xr, Ws)


def _final_body(x_ref, w_ref, b_ref, o_ref):
    t = pl.program_id(1)
    part = lax.dot_general(x_ref[0].astype(BF), w_ref[...],
                           (((1,), (0,)), ((), ())),
                           preferred_element_type=jnp.float32) + b_ref[...]
    m = jnp.max(part, axis=0, keepdims=True)
    prev = jnp.where(t == 0, NEG, o_ref[0])
    o_ref[0] = jnp.maximum(prev, m)


def _final_max(x123r, W6, b6):
    Bn, Nn, Cc = x123r.shape
    return pl.pallas_call(
        _final_body,
        grid=(Bn, Nn // TNF),
        in_specs=[
            pl.BlockSpec((1, TNF, Cc), lambda b, t: (b, t, 0)),
            pl.BlockSpec((Cc, 1024), lambda b, t: (0, 0)),
            pl.BlockSpec((1, 1024), lambda b, t: (0, 0)),
        ],
        out_specs=pl.BlockSpec((1, 1, 1024), lambda b, t: (b, 0, 0)),
        out_shape=jax.ShapeDtypeStruct((Bn, 1, 1024), jnp.float32),
    )(x123r, W6.T.astype(BF), b6.reshape(1, 1024))


def kernel(x, W1, b1, W2, b2, W3, b3, W4, b4, W5, b5, W6, b6):
    Bn = x.shape[0]
    xT1 = jnp.pad(x, ((0, 0), (0, 5), (0, 0)))
    xr1 = jnp.transpose(xT1, (0, 2, 1))
    tab1 = jnp.pad(xr1, ((0, 0), (0, 0), (0, YW - 8)))
    tab1 = tab1.reshape(Bn * NPTS, YW)
    W1e = jnp.zeros((64, 16), x.dtype)
    W1e = W1e.at[:, 0:3].set(W1[:, 0:3]).at[:, 8:11].set(W1[:, 3:6])

    x1 = _stage(xr1, xT1, tab1, [(W1e, b1), (W2, b2)])

    tab2 = jnp.pad(x1, ((0, 0), (0, 0), (0, YW - 64))).reshape(Bn * NPTS, YW)
    x2 = _stage(x1, jnp.transpose(x1, (0, 2, 1)), tab2, [(W3, b3), (W4, b4)])

    tab3 = jnp.pad(x2, ((0, 0), (0, 0), (0, YW - 64))).reshape(Bn * NPTS, YW)
    x3 = _stage(x2, jnp.transpose(x2, (0, 2, 1)), tab3, [(W5, b5)])

    x123r = jnp.concatenate([x1, x2, x3], axis=2)
    x123 = jnp.transpose(x123r, (0, 2, 1))
    x5 = _final_max(x123r, W6, b6)
    return (x123, jnp.transpose(x5, (0, 2, 1)))


# speedup vs baseline: 1.3159x; 1.3159x over previous
import functools

import jax
import jax.numpy as jnp
from jax import lax
from jax.experimental import pallas as pl
from jax.experimental.pallas import tpu as pltpu
from jax.experimental.pallas import tpu_sc as plsc

KNN = 20
NPTS = 4096
TN = 256
TNB = 128
TNF = 512
CH = 32
CHC = 128
YW = 128
NWORK = 32
NEG = -3.0e38
BF = jnp.bfloat16


NCL = 128
NSG = NPTS // NCL
RND = 5


def _topk_body(xT_ref, xr_ref, idx_ref, D_ref, cand_ref, cpos_ref):
    b = pl.program_id(0)
    xT = xT_ref[0]
    xr = xr_ref[0]
    G = lax.dot_general(xr.astype(BF), xT.astype(BF), (((1,), (0,)), ((), ())),
                        preferred_element_type=jnp.float32)
    nb = jnp.sum(xT * xT, axis=0, keepdims=True)
    nr = jnp.sum(xr * xr, axis=1, keepdims=True)
    D_ref[...] = 2.0 * G - nr - nb

    gcol = (lax.broadcasted_iota(jnp.int32, (TN, NSG, NCL), 1) * NCL
            + lax.broadcasted_iota(jnp.int32, (TN, NSG, NCL), 2))
    rowsK = lax.broadcasted_iota(jnp.int32, (KNN, TN), 0)
    lanes = lax.broadcasted_iota(jnp.int32, (TN, NPTS), 1)

    def rbody(r, mth):
        D3 = D_ref[...].reshape(TN, NSG, NCL)
        masked = jnp.where(D3 < mth[:, None, :], D3, NEG)
        cm = jnp.max(masked, axis=1)
        pos = jnp.min(jnp.where(masked == cm[:, None, :], gcol, NPTS),
                      axis=1)
        cand_ref[pl.ds(r, 1)] = cm[None]
        cpos_ref[pl.ds(r, 1)] = pos[None]
        return cm

    mth0 = jnp.full((TN, NCL), 3.0e38, jnp.float32)
    mth = lax.fori_loop(0, RND, rbody, mth0)
    g = jnp.max(mth, axis=1, keepdims=True)

    idx_ref[0] = jnp.zeros((KNN, TN), jnp.int32)
    cnd0 = cand_ref[...]
    m0 = jnp.max(jnp.max(cnd0, axis=0), axis=1, keepdims=True)

    def fstep(t, carry):
        m, _ = carry
        cnd = cand_ref[...]
        cp = cpos_ref[...]
        col = jnp.min(jnp.min(jnp.where(cnd == m[None], cp, NPTS), axis=0),
                      axis=1)
        idx_ref[0] = jnp.where(rowsK == t, (col + b * NPTS)[None, :],
                               idx_ref[0])
        nxt = jnp.max(jnp.max(jnp.where(cnd < m[None], cnd, NEG), axis=0),
                      axis=1, keepdims=True)
        return nxt, m

    _, v20 = lax.fori_loop(0, KNN, fstep, (m0, m0))

    @pl.when(jnp.any(v20 < g))
    def _fallback():
        m00 = jnp.max(D_ref[...], axis=1, keepdims=True)

        def step(t, m):
            D = D_ref[...]
            col = jnp.min(jnp.where(D == m, lanes, NPTS), axis=1)
            idx_ref[0] = jnp.where(rowsK == t, (col + b * NPTS)[None, :],
                                   idx_ref[0])
            return jnp.max(jnp.where(D < m, D, NEG), axis=1, keepdims=True)

        lax.fori_loop(0, KNN, step, m00)


def _topk(xr, xT):
    Bn, Nn, C = xr.shape
    return pl.pallas_call(
        _topk_body,
        grid=(Bn, Nn // TN),
        in_specs=[
            pl.BlockSpec((1, C, Nn), lambda b, t: (b, 0, 0)),
            pl.BlockSpec((1, TN, C), lambda b, t: (b, t, 0)),
        ],
        out_specs=pl.BlockSpec((1, KNN, TN), lambda b, t: (b, 0, t)),
        out_shape=jax.ShapeDtypeStruct((Bn, KNN, Nn), jnp.int32),
        scratch_shapes=[pltpu.VMEM((TN, Nn), jnp.float32),
                        pltpu.VMEM((RND, TN, NCL), jnp.float32),
                        pltpu.VMEM((RND, TN, NCL), jnp.int32)],
    )(xT, xr)


def _sc_gather(table, idxg, Bn, Nn):
    P = Bn * Nn
    ppw = P // NWORK
    nslab = ppw // CHC

    mesh = plsc.VectorSubcoreMesh(core_axis_name="c", subcore_axis_name="s")

    @functools.partial(
        pl.kernel,
        mesh=mesh,
        out_type=jax.ShapeDtypeStruct((Bn, KNN, Nn, YW), jnp.float32),
        scratch_types=[
            pltpu.VMEM((KNN, CHC), jnp.int32),
            pltpu.VMEM((KNN, CH, YW), jnp.float32),
            pltpu.SemaphoreType.DMA,
        ],
    )
    def sc_body(tab_hbm, idx_hbm, out_hbm, idxv, rows, sem):
        wid = lax.axis_index("s") * 2 + lax.axis_index("c")
        p0w = wid * ppw
        b = p0w // Nn

        def slab(ci, _):
            jb = p0w + ci * CHC - b * Nn
            pltpu.sync_copy(idx_hbm.at[b, :, pl.ds(jb, CHC)], idxv)

            def sub(h, _):
                jc = jb + h * CH
                cps = [pltpu.async_copy(
                           tab_hbm.at[idxv.at[t, pl.ds(h * CH, CH)]],
                           rows.at[t], sem)
                       for t in range(KNN)]
                for cp in cps:
                    cp.wait()
                pltpu.sync_copy(rows, out_hbm.at[b, :, pl.ds(jc, CH)])
                return 0

            lax.fori_loop(0, CHC // CH, sub, 0)
            return 0

        lax.fori_loop(0, nslab, slab, 0)

    return sc_body(table, idxg)


def _edge_body2(E_ref, xr_ref, W1_ref, b1_ref, W2_ref, b2_ref, o_ref):
    C = xr_ref.shape[2]
    xj = E_ref[0].reshape(KNN * TNB, YW)[:, :C]
    xi = jnp.broadcast_to(xr_ref[0][None], (KNN, TNB, C))
    xi = xi.reshape(KNN * TNB, C)
    feat = jnp.concatenate([xj - xi, xi], axis=1).astype(BF)
    h = lax.dot_general(feat, W1_ref[...], (((1,), (0,)), ((), ())),
                        preferred_element_type=jnp.float32) + b1_ref[...]
    h = lax.dot_general(h.astype(BF), W2_ref[...], (((1,), (0,)), ((), ())),
                        preferred_element_type=jnp.float32) + b2_ref[...]
    o_ref[0] = jnp.max(h.reshape(KNN, TNB, 64), axis=0)


def _edge_body1(E_ref, xr_ref, W1_ref, b1_ref, o_ref):
    C = xr_ref.shape[2]
    xj = E_ref[0].reshape(KNN * TNB, YW)[:, :C]
    xi = jnp.broadcast_to(xr_ref[0][None], (KNN, TNB, C))
    xi = xi.reshape(KNN * TNB, C)
    feat = jnp.concatenate([xj - xi, xi], axis=1).astype(BF)
    h = lax.dot_general(feat, W1_ref[...], (((1,), (0,)), ((), ())),
                        preferred_element_type=jnp.float32) + b1_ref[...]
    o_ref[0] = jnp.max(h.reshape(KNN, TNB, 64), axis=0)


def _edgeconv(E, xr, Ws):
    Bn, Nn, C = xr.shape
    C2 = 2 * C
    wspecs = []
    wargs = []
    for (W, b) in Ws:
        cdim = W.shape[1]
        wspecs += [pl.BlockSpec((cdim, 64), lambda b_, t: (0, 0)),
                   pl.BlockSpec((1, 64), lambda b_, t: (0, 0))]
        wargs += [W.T.astype(BF), b.reshape(1, 64)]
    body = _edge_body2 if len(Ws) == 2 else _edge_body1
    return pl.pallas_call(
        body,
        grid=(Bn, Nn // TNB),
        in_specs=[
            pl.BlockSpec((1, KNN, TNB, YW), lambda b, t: (b, 0, t, 0)),
            pl.BlockSpec((1, TNB, C), lambda b, t: (b, t, 0)),
        ] + wspecs,
        out_specs=pl.BlockSpec((1, TNB, 64), lambda b, t: (b, t, 0)),
        out_shape=jax.ShapeDtypeStruct((Bn, Nn, 64), jnp.float32),
    )(E, xr, *wargs)


def _stage(xr, xT, table, Ws):
    Bn, Nn, _ = xr.shape
    idx = _topk(xr, xT)
    E = _sc_gather(table, idx, Bn, Nn)
    return _edgeconv(E, xr, Ws)


def _final_body(x_ref, w_ref, b_ref, o_ref):
    t = pl.program_id(1)
    part = lax.dot_general(x_ref[0].astype(BF), w_ref[...],
                           (((1,), (0,)), ((), ())),
                           preferred_element_type=jnp.float32) + b_ref[...]
    m = jnp.max(part, axis=0, keepdims=True)
    prev = jnp.where(t == 0, NEG, o_ref[0])
    o_ref[0] = jnp.maximum(prev, m)


def _final_max(x123r, W6, b6):
    Bn, Nn, Cc = x123r.shape
    return pl.pallas_call(
        _final_body,
        grid=(Bn, Nn // TNF),
        in_specs=[
            pl.BlockSpec((1, TNF, Cc), lambda b, t: (b, t, 0)),
            pl.BlockSpec((Cc, 1024), lambda b, t: (0, 0)),
            pl.BlockSpec((1, 1024), lambda b, t: (0, 0)),
        ],
        out_specs=pl.BlockSpec((1, 1, 1024), lambda b, t: (b, 0, 0)),
        out_shape=jax.ShapeDtypeStruct((Bn, 1, 1024), jnp.float32),
    )(x123r, W6.T.astype(BF), b6.reshape(1, 1024))


def kernel(x, W1, b1, W2, b2, W3, b3, W4, b4, W5, b5, W6, b6):
    Bn = x.shape[0]
    xT1 = jnp.pad(x, ((0, 0), (0, 5), (0, 0)))
    xr1 = jnp.transpose(xT1, (0, 2, 1))
    tab1 = jnp.pad(xr1, ((0, 0), (0, 0), (0, YW - 8)))
    tab1 = tab1.reshape(Bn * NPTS, YW)
    W1e = jnp.zeros((64, 16), x.dtype)
    W1e = W1e.at[:, 0:3].set(W1[:, 0:3]).at[:, 8:11].set(W1[:, 3:6])

    x1 = _stage(xr1, xT1, tab1, [(W1e, b1), (W2, b2)])

    tab2 = jnp.pad(x1, ((0, 0), (0, 0), (0, YW - 64))).reshape(Bn * NPTS, YW)
    x2 = _stage(x1, jnp.transpose(x1, (0, 2, 1)), tab2, [(W3, b3), (W4, b4)])

    tab3 = jnp.pad(x2, ((0, 0), (0, 0), (0, YW - 64))).reshape(Bn * NPTS, YW)
    x3 = _stage(x2, jnp.transpose(x2, (0, 2, 1)), tab3, [(W5, b5)])

    x123r = jnp.concatenate([x1, x2, x3], axis=2)
    x123 = jnp.transpose(x123r, (0, 2, 1))
    x5 = _final_max(x123r, W6, b6)
    return (x123, jnp.transpose(x5, (0, 2, 1)))


# speedup vs baseline: 1.3342x; 1.0139x over previous
import functools

import jax
import jax.numpy as jnp
from jax import lax
from jax.experimental import pallas as pl
from jax.experimental.pallas import tpu as pltpu
from jax.experimental.pallas import tpu_sc as plsc

KNN = 20
NPTS = 4096
TN = 256
TNB = 128
TNF = 512
CH = 32
CHC = 128
YW = 128
NWORK = 32
NEG = -3.0e38
BF = jnp.bfloat16


NCL = 128
NSG = NPTS // NCL
RND = 5


def _topk_body(xT_ref, xr_ref, idx_ref, D_ref, cand_ref, cpos_ref):
    b = pl.program_id(0)
    xT = xT_ref[0]
    xr = xr_ref[0]
    G = lax.dot_general(xr.astype(BF), xT.astype(BF), (((1,), (0,)), ((), ())),
                        preferred_element_type=jnp.float32)
    nb = jnp.sum(xT * xT, axis=0, keepdims=True)
    nr = jnp.sum(xr * xr, axis=1, keepdims=True)
    D_ref[...] = 2.0 * G - nr - nb

    gcol = (lax.broadcasted_iota(jnp.int32, (TN, NSG, NCL), 1) * NCL
            + lax.broadcasted_iota(jnp.int32, (TN, NSG, NCL), 2))
    rowsK = lax.broadcasted_iota(jnp.int32, (KNN, TN), 0)
    lanes = lax.broadcasted_iota(jnp.int32, (TN, NPTS), 1)

    def rbody(r, mth):
        D3 = D_ref[...].reshape(TN, NSG, NCL)
        masked = jnp.where(D3 < mth[:, None, :], D3, NEG)
        cm = jnp.max(masked, axis=1)
        pos = jnp.min(jnp.where(masked == cm[:, None, :], gcol, NPTS),
                      axis=1)
        cand_ref[pl.ds(r, 1)] = cm[None]
        cpos_ref[pl.ds(r, 1)] = pos[None]
        return cm

    mth0 = jnp.full((TN, NCL), 3.0e38, jnp.float32)
    mth = lax.fori_loop(0, RND, rbody, mth0)
    g = jnp.max(mth, axis=1, keepdims=True)

    idx_ref[0] = jnp.zeros((KNN, TN), jnp.int32)
    cnd0 = cand_ref[...]
    m0 = jnp.max(jnp.max(cnd0, axis=0), axis=1, keepdims=True)

    def fstep(t, carry):
        m, _ = carry
        cnd = cand_ref[...]
        cp = cpos_ref[...]
        col = jnp.min(jnp.min(jnp.where(cnd == m[None], cp, NPTS), axis=0),
                      axis=1)
        idx_ref[0] = jnp.where(rowsK == t, (col + b * NPTS)[None, :],
                               idx_ref[0])
        nxt = jnp.max(jnp.max(jnp.where(cnd < m[None], cnd, NEG), axis=0),
                      axis=1, keepdims=True)
        return nxt, m

    _, v20 = lax.fori_loop(0, KNN, fstep, (m0, m0))

    @pl.when(jnp.any(v20 < g))
    def _fallback():
        m00 = jnp.max(D_ref[...], axis=1, keepdims=True)

        def step(t, m):
            D = D_ref[...]
            col = jnp.min(jnp.where(D == m, lanes, NPTS), axis=1)
            idx_ref[0] = jnp.where(rowsK == t, (col + b * NPTS)[None, :],
                                   idx_ref[0])
            return jnp.max(jnp.where(D < m, D, NEG), axis=1, keepdims=True)

        lax.fori_loop(0, KNN, step, m00)


def _topk(xr, xT):
    Bn, Nn, C = xr.shape
    return pl.pallas_call(
        _topk_body,
        grid=(Bn, Nn // TN),
        in_specs=[
            pl.BlockSpec((1, C, Nn), lambda b, t: (b, 0, 0)),
            pl.BlockSpec((1, TN, C), lambda b, t: (b, t, 0)),
        ],
        out_specs=pl.BlockSpec((1, KNN, TN), lambda b, t: (b, 0, t)),
        out_shape=jax.ShapeDtypeStruct((Bn, KNN, Nn), jnp.int32),
        scratch_shapes=[pltpu.VMEM((TN, Nn), jnp.float32),
                        pltpu.VMEM((RND, TN, NCL), jnp.float32),
                        pltpu.VMEM((RND, TN, NCL), jnp.int32)],
    )(xT, xr)


def _sc_gather(table, idxg, Bn, Nn):
    P = Bn * Nn
    ppw = P // NWORK
    nslab = ppw // CHC

    mesh = plsc.VectorSubcoreMesh(core_axis_name="c", subcore_axis_name="s")

    @functools.partial(
        pl.kernel,
        mesh=mesh,
        out_type=jax.ShapeDtypeStruct((Bn, KNN, Nn, YW), jnp.float32),
        scratch_types=[
            pltpu.VMEM((KNN, CHC), jnp.int32),
            pltpu.VMEM((KNN, CH, YW), jnp.float32),
            pltpu.SemaphoreType.DMA,
        ],
    )
    def sc_body(tab_hbm, idx_hbm, out_hbm, idxv, rows, sem):
        wid = lax.axis_index("s") * 2 + lax.axis_index("c")
        p0w = wid * ppw
        b = p0w // Nn

        def slab(ci, _):
            jb = p0w + ci * CHC - b * Nn
            pltpu.sync_copy(idx_hbm.at[b, :, pl.ds(jb, CHC)], idxv)

            def sub(h, _):
                jc = jb + h * CH
                cps = [pltpu.async_copy(
                           tab_hbm.at[idxv.at[t, pl.ds(h * CH, CH)]],
                           rows.at[t], sem)
                       for t in range(KNN)]
                for cp in cps:
                    cp.wait()
                pltpu.sync_copy(rows, out_hbm.at[b, :, pl.ds(jc, CH)])
                return 0

            lax.fori_loop(0, CHC // CH, sub, 0)
            return 0

        lax.fori_loop(0, nslab, slab, 0)

    return sc_body(table, idxg)


def _edge_body2(E_ref, xr_ref, W1_ref, b1_ref, W2_ref, b2_ref, o_ref):
    C = xr_ref.shape[2]
    xj = E_ref[0].reshape(KNN * TNB, YW)[:, :C]
    xi = jnp.broadcast_to(xr_ref[0][None], (KNN, TNB, C))
    xi = xi.reshape(KNN * TNB, C)
    feat = jnp.concatenate([xj - xi, xi], axis=1).astype(BF)
    h = lax.dot_general(feat, W1_ref[...], (((1,), (0,)), ((), ())),
                        preferred_element_type=jnp.float32) + b1_ref[...]
    h = lax.dot_general(h.astype(BF), W2_ref[...], (((1,), (0,)), ((), ())),
                        preferred_element_type=jnp.float32) + b2_ref[...]
    o_ref[0] = jnp.max(h.reshape(KNN, TNB, 64), axis=0)


def _edge_body1(E_ref, xr_ref, W1_ref, b1_ref, o_ref):
    C = xr_ref.shape[2]
    xj = E_ref[0].reshape(KNN * TNB, YW)[:, :C]
    xi = jnp.broadcast_to(xr_ref[0][None], (KNN, TNB, C))
    xi = xi.reshape(KNN * TNB, C)
    feat = jnp.concatenate([xj - xi, xi], axis=1).astype(BF)
    h = lax.dot_general(feat, W1_ref[...], (((1,), (0,)), ((), ())),
                        preferred_element_type=jnp.float32) + b1_ref[...]
    o_ref[0] = jnp.max(h.reshape(KNN, TNB, 64), axis=0)


def _edgeconv(E, xr, Ws):
    Bn, Nn, C = xr.shape
    C2 = 2 * C
    wspecs = []
    wargs = []
    for (W, b) in Ws:
        cdim = W.shape[1]
        wspecs += [pl.BlockSpec((cdim, 64), lambda b_, t: (0, 0)),
                   pl.BlockSpec((1, 64), lambda b_, t: (0, 0))]
        wargs += [W.T.astype(BF), b.reshape(1, 64)]
    body = _edge_body2 if len(Ws) == 2 else _edge_body1
    return pl.pallas_call(
        body,
        grid=(Bn, Nn // TNB),
        in_specs=[
            pl.BlockSpec((1, KNN, TNB, YW), lambda b, t: (b, 0, t, 0)),
            pl.BlockSpec((1, TNB, C), lambda b, t: (b, t, 0)),
        ] + wspecs,
        out_specs=pl.BlockSpec((1, TNB, 64), lambda b, t: (b, t, 0)),
        out_shape=jax.ShapeDtypeStruct((Bn, Nn, 64), jnp.float32),
    )(E, xr, *wargs)


def _stage(xr, xT, table, Ws):
    Bn, Nn, _ = xr.shape
    idx = _topk(xr, xT)
    h = Nn // 2
    outs = []
    for s in range(2):
        idx_h = lax.slice_in_dim(idx, s * h, (s + 1) * h, axis=2)
        E = _sc_gather(table, idx_h, Bn, h)
        xr_h = lax.slice_in_dim(xr, s * h, (s + 1) * h, axis=1)
        outs.append(_edgeconv(E, xr_h, Ws))
    return jnp.concatenate(outs, axis=1)


def _final_body(x_ref, w_ref, b_ref, o_ref):
    t = pl.program_id(1)
    part = lax.dot_general(x_ref[0].astype(BF), w_ref[...],
                           (((1,), (0,)), ((), ())),
                           preferred_element_type=jnp.float32) + b_ref[...]
    m = jnp.max(part, axis=0, keepdims=True)
    prev = jnp.where(t == 0, NEG, o_ref[0])
    o_ref[0] = jnp.maximum(prev, m)


def _final_max(x123r, W6, b6):
    Bn, Nn, Cc = x123r.shape
    return pl.pallas_call(
        _final_body,
        grid=(Bn, Nn // TNF),
        in_specs=[
            pl.BlockSpec((1, TNF, Cc), lambda b, t: (b, t, 0)),
            pl.BlockSpec((Cc, 1024), lambda b, t: (0, 0)),
            pl.BlockSpec((1, 1024), lambda b, t: (0, 0)),
        ],
        out_specs=pl.BlockSpec((1, 1, 1024), lambda b, t: (b, 0, 0)),
        out_shape=jax.ShapeDtypeStruct((Bn, 1, 1024), jnp.float32),
    )(x123r, W6.T.astype(BF), b6.reshape(1, 1024))


def kernel(x, W1, b1, W2, b2, W3, b3, W4, b4, W5, b5, W6, b6):
    Bn = x.shape[0]
    xT1 = jnp.pad(x, ((0, 0), (0, 5), (0, 0)))
    xr1 = jnp.transpose(xT1, (0, 2, 1))
    tab1 = jnp.pad(xr1, ((0, 0), (0, 0), (0, YW - 8)))
    tab1 = tab1.reshape(Bn * NPTS, YW)
    W1e = jnp.zeros((64, 16), x.dtype)
    W1e = W1e.at[:, 0:3].set(W1[:, 0:3]).at[:, 8:11].set(W1[:, 3:6])

    x1 = _stage(xr1, xT1, tab1, [(W1e, b1), (W2, b2)])

    tab2 = jnp.pad(x1, ((0, 0), (0, 0), (0, YW - 64))).reshape(Bn * NPTS, YW)
    x2 = _stage(x1, jnp.transpose(x1, (0, 2, 1)), tab2, [(W3, b3), (W4, b4)])

    tab3 = jnp.pad(x2, ((0, 0), (0, 0), (0, YW - 64))).reshape(Bn * NPTS, YW)
    x3 = _stage(x2, jnp.transpose(x2, (0, 2, 1)), tab3, [(W5, b5)])

    x123r = jnp.concatenate([x1, x2, x3], axis=2)
    x123 = jnp.transpose(x123r, (0, 2, 1))
    x5 = _final_max(x123r, W6, b6)
    return (x123, jnp.transpose(x5, (0, 2, 1)))
